# baseline (speedup 1.0000x reference)
import functools

import numpy as np
import jax
import jax.numpy as jnp
from jax import lax
from jax.experimental import pallas as pl
from jax.experimental.pallas import tpu as pltpu
from jax.experimental.pallas import tpu_sc as plsc

_N = 10000
_NPAD = 10240
_E = 160000
_D = 256
_HALF = 128
_L = 4
_ALPHA = 0.1
_THETA = 0.5
_NC = 2
_NS = 16
_G = 128
_GS = 64
_SPG = _G // _GS

_EPT = 10240
_EP = _NS * _EPT
_KP = _EPT // _G
_RPT = _NPAD // _NS

_mesh = plsc.VectorSubcoreMesh(core_axis_name="c", subcore_axis_name="s")


def _fill2d(ref, rows, cols, value):
    vals = jnp.full((16,), value, jnp.float32)

    def body(i, _):
        for j in range(cols // 16):
            ref[i, pl.ds(j * 16, 16)] = vals
        return 0

    lax.fori_loop(0, rows, body, 0)


@functools.partial(
    pl.kernel,
    out_type=jax.ShapeDtypeStruct((_NC, _NPAD, _HALF), jnp.float32),
    mesh=_mesh,
    scratch_types=[
        pltpu.VMEM((_KP, _G), jnp.int32),
        pltpu.VMEM((_KP * _SPG, _GS), jnp.int32),
        pltpu.VMEM((_G, _HALF), jnp.float32),
        pltpu.VMEM((_G, _HALF), jnp.float32),
        pltpu.VMEM_SHARED((_NPAD, _HALF), jnp.float32),
        pltpu.SemaphoreType.DMA,
        pltpu.SemaphoreType.DMA,
    ],
)
def _prop_sc(hs_hbm, src_hbm, dst_hbm, out_hbm, src_v, dst_v, rows_a,
             rows_b, acc_sh, sem_a, sem_b):
    cid = lax.axis_index("c")
    sid = lax.axis_index("s")
    wid = cid * _NS + sid
    _fill2d(rows_a, _GS, _HALF, 0.0)
    for r in range(_RPT // _GS):
        pltpu.sync_copy(rows_a.at[pl.ds(0, _GS)],
                        acc_sh.at[pl.ds(sid * _RPT + r * _GS, _GS)])
    plsc.subcore_barrier()
    pltpu.sync_copy(src_hbm.at[wid], src_v)
    pltpu.sync_copy(dst_hbm.at[sid], dst_v)

    def body(j, _):
        pltpu.async_copy(hs_hbm.at[src_v.at[j]], rows_a, sem_a).wait()
        for q in range(_SPG):
            pltpu.sync_copy(rows_a.at[pl.ds(q * _GS, _GS)],
                            acc_sh.at[dst_v.at[_SPG * j + q]], add=True)
        return 0

    lax.fori_loop(0, _KP, body, 0)
    plsc.subcore_barrier()
    for r in range(_RPT // _GS):
        sl = pl.ds(sid * _RPT + r * _GS, _GS)
        pltpu.sync_copy(acc_sh.at[sl], out_hbm.at[cid, sl])


_RT = 1024


def _prep_body(x_ref, w_ref, b_ref, d0_ref, d1_ref, x0_ref, hs0_ref, hs1_ref):
    x0 = jnp.dot(x_ref[...], w_ref[...],
                 preferred_element_type=jnp.float32) + b_ref[...]
    deg = 0.5 * (d0_ref[...][:, 0:1] + d1_ref[...][:, 0:1]) + 1.0
    dinv = lax.rsqrt(deg)
    x0_ref[...] = x0
    hs = dinv * x0
    hs0_ref[...] = hs[:, :_HALF]
    hs1_ref[...] = hs[:, _HALF:]


def _tc_prep(x_p, wt, b2, d0, d1):
    grid = (_NPAD // _RT,)
    return pl.pallas_call(
        _prep_body,
        grid=grid,
        in_specs=[
            pl.BlockSpec((_RT, _D), lambda i: (i, 0)),
            pl.BlockSpec((_D, _D), lambda i: (0, 0)),
            pl.BlockSpec((1, _D), lambda i: (0, 0)),
            pl.BlockSpec((_RT, _HALF), lambda i: (i, 0)),
            pl.BlockSpec((_RT, _HALF), lambda i: (i, 0)),
        ],
        out_specs=[
            pl.BlockSpec((_RT, _D), lambda i: (i, 0)),
            pl.BlockSpec((_RT, _HALF), lambda i: (i, 0)),
            pl.BlockSpec((_RT, _HALF), lambda i: (i, 0)),
        ],
        out_shape=[
            jax.ShapeDtypeStruct((_NPAD, _D), jnp.float32),
            jax.ShapeDtypeStruct((_NPAD, _HALF), jnp.float32),
            jax.ShapeDtypeStruct((_NPAD, _HALF), jnp.float32),
        ],
    )(x_p, wt, b2, d0, d1)


def _update_body(beta, relu, a0_ref, a1_ref, x0_ref, h_ref, d0_ref, d1_ref,
                 w_ref, hn_ref, hs0_ref, hs1_ref):
    deg = 0.5 * (d0_ref[...][:, 0:1] + d1_ref[...][:, 0:1]) + 1.0
    dinv = lax.rsqrt(deg)
    h = h_ref[...]
    aggf = jnp.concatenate([a0_ref[...], a1_ref[...]], axis=1)
    agg = dinv * aggf + (dinv * dinv) * h
    t = (1.0 - _ALPHA) * agg + _ALPHA * x0_ref[...]
    raw = (1.0 - beta) * t + beta * jnp.dot(
        t, w_ref[...], preferred_element_type=jnp.float32)
    h2 = raw + h
    if relu:
        h2 = jnp.maximum(h2, 0.0)
    hn_ref[...] = h2
    hs = dinv * h2
    hs0_ref[...] = hs[:, :_HALF]
    hs1_ref[...] = hs[:, _HALF:]


def _tc_update(a0, a1, x0, h, d0, d1, w, beta, relu):
    grid = (_NPAD // _RT,)
    return pl.pallas_call(
        functools.partial(_update_body, beta, relu),
        grid=grid,
        in_specs=[
            pl.BlockSpec((_RT, _HALF), lambda i: (i, 0)),
            pl.BlockSpec((_RT, _HALF), lambda i: (i, 0)),
            pl.BlockSpec((_RT, _D), lambda i: (i, 0)),
            pl.BlockSpec((_RT, _D), lambda i: (i, 0)),
            pl.BlockSpec((_RT, _HALF), lambda i: (i, 0)),
            pl.BlockSpec((_RT, _HALF), lambda i: (i, 0)),
            pl.BlockSpec((_D, _D), lambda i: (0, 0)),
        ],
        out_specs=[
            pl.BlockSpec((_RT, _D), lambda i: (i, 0)),
            pl.BlockSpec((_RT, _HALF), lambda i: (i, 0)),
            pl.BlockSpec((_RT, _HALF), lambda i: (i, 0)),
        ],
        out_shape=[
            jax.ShapeDtypeStruct((_NPAD, _D), jnp.float32),
            jax.ShapeDtypeStruct((_NPAD, _HALF), jnp.float32),
            jax.ShapeDtypeStruct((_NPAD, _HALF), jnp.float32),
        ],
    )(a0, a1, x0, h, d0, d1, w)


def kernel(x, edge_index, W_proj, b_proj, W_convs):
    src = edge_index[0]
    dst = edge_index[1]
    pad_p = _EP - _E
    srcp = jnp.concatenate([src, jnp.zeros((pad_p,), jnp.int32)])
    dstp = jnp.concatenate([dst, jnp.full((pad_p,), _NPAD - 1, jnp.int32)])
    src3 = jnp.concatenate([srcp, srcp + _NPAD]).reshape(_NC * _NS, _KP, _G)
    dst3 = dstp.reshape(_NS, _KP * _SPG, _GS)
    ones_flat = jnp.ones((_NC * _NPAD, _HALF), jnp.float32)
    deg_parts = _prop_sc(ones_flat, src3, dst3)
    d0, d1 = deg_parts[0], deg_parts[1]

    x_p = jnp.pad(x, ((0, _NPAD - _N), (0, 0)))
    x0, hs0, hs1 = _tc_prep(x_p, W_proj.T, b_proj.reshape(1, _D), d0, d1)
    h = x0
    for i in range(_L):
        hs_flat = jnp.concatenate([hs0, hs1], axis=0)
        agg = _prop_sc(hs_flat, src3, dst3)
        beta = float(np.log(_THETA / (i + 1) + 1.0))
        h, hs0, hs1 = _tc_update(agg[0], agg[1], x0, h, d0, d1, W_convs[i],
                                 beta, relu=(i < _L - 1))
    return h[:_N]


# speedup vs baseline: 1.0927x; 1.0927x over previous
import functools

import numpy as np
import jax
import jax.numpy as jnp
from jax import lax
from jax.experimental import pallas as pl
from jax.experimental.pallas import tpu as pltpu
from jax.experimental.pallas import tpu_sc as plsc

_N = 10000
_NPAD = 10240
_E = 160000
_D = 256
_HALF = 128
_L = 4
_ALPHA = 0.1
_THETA = 0.5
_NC = 2
_NS = 16
_G = 128
_GS = 64
_SPG = _G // _GS

_EPT = 10240
_EP = _NS * _EPT
_KP = _EPT // _G
_RPT = _NPAD // _NS
_ED = _NC * _NS * 5120
_KD = 5120 // _GS

_mesh = plsc.VectorSubcoreMesh(core_axis_name="c", subcore_axis_name="s")


def _fill2d(ref, rows, cols, value):
    vals = jnp.full((16,), value, jnp.float32)

    def body(i, _):
        for j in range(cols // 16):
            ref[i, pl.ds(j * 16, 16)] = vals
        return 0

    lax.fori_loop(0, rows, body, 0)


@functools.partial(
    pl.kernel,
    out_type=jax.ShapeDtypeStruct((_NC, _NPAD, _HALF), jnp.float32),
    mesh=_mesh,
    scratch_types=[
        pltpu.VMEM((_KD, _GS), jnp.int32),
        pltpu.VMEM((_GS, _HALF), jnp.float32),
        pltpu.VMEM_SHARED((_NPAD, _HALF), jnp.float32),
    ],
)
def _deg_sc(dst_hbm, out_hbm, idx_v, ones_v, acc_sh):
    cid = lax.axis_index("c")
    sid = lax.axis_index("s")
    wid = cid * _NS + sid
    _fill2d(ones_v, _GS, _HALF, 0.0)
    for r in range(_RPT // _GS):
        pltpu.sync_copy(ones_v,
                        acc_sh.at[pl.ds(sid * _RPT + r * _GS, _GS)])
    _fill2d(ones_v, _GS, _HALF, 1.0)
    plsc.subcore_barrier()
    pltpu.sync_copy(dst_hbm.at[wid], idx_v)

    def body(j, _):
        pltpu.sync_copy(ones_v, acc_sh.at[idx_v.at[j]], add=True)
        return 0

    lax.fori_loop(0, _KD, body, 0)
    plsc.subcore_barrier()
    for r in range(_RPT // _GS):
        sl = pl.ds(sid * _RPT + r * _GS, _GS)
        pltpu.sync_copy(acc_sh.at[sl], out_hbm.at[cid, sl])


@functools.partial(
    pl.kernel,
    out_type=jax.ShapeDtypeStruct((_NC, _NPAD, _HALF), jnp.float32),
    mesh=_mesh,
    scratch_types=[
        pltpu.VMEM((_KP, _G), jnp.int32),
        pltpu.VMEM((_KP * _SPG, _GS), jnp.int32),
        pltpu.VMEM((_G, _HALF), jnp.float32),
        pltpu.VMEM((_G, _HALF), jnp.float32),
        pltpu.VMEM_SHARED((_NPAD, _HALF), jnp.float32),
        pltpu.SemaphoreType.DMA,
        pltpu.SemaphoreType.DMA,
    ],
)
def _prop_sc(hs_hbm, src_hbm, dst_hbm, out_hbm, src_v, dst_v, rows_a,
             rows_b, acc_sh, sem_a, sem_b):
    cid = lax.axis_index("c")
    sid = lax.axis_index("s")
    wid = cid * _NS + sid
    _fill2d(rows_a, _GS, _HALF, 0.0)
    for r in range(_RPT // _GS):
        pltpu.sync_copy(rows_a.at[pl.ds(0, _GS)],
                        acc_sh.at[pl.ds(sid * _RPT + r * _GS, _GS)])
    plsc.subcore_barrier()
    pltpu.sync_copy(src_hbm.at[wid], src_v)
    pltpu.sync_copy(dst_hbm.at[sid], dst_v)

    def body(j, _):
        pltpu.async_copy(hs_hbm.at[src_v.at[j]], rows_a, sem_a).wait()
        for q in range(_SPG):
            pltpu.sync_copy(rows_a.at[pl.ds(q * _GS, _GS)],
                            acc_sh.at[dst_v.at[_SPG * j + q]], add=True)
        return 0

    lax.fori_loop(0, _KP, body, 0)
    plsc.subcore_barrier()
    for r in range(_RPT // _GS):
        sl = pl.ds(sid * _RPT + r * _GS, _GS)
        pltpu.sync_copy(acc_sh.at[sl], out_hbm.at[cid, sl])


_RT = 1024


def _prep_body(x_ref, w_ref, b_ref, d0_ref, d1_ref, x0_ref, hs0_ref, hs1_ref):
    x0 = jnp.dot(x_ref[...], w_ref[...],
                 preferred_element_type=jnp.float32) + b_ref[...]
    deg = d0_ref[...][:, 0:1] + d1_ref[...][:, 0:1] + 1.0
    dinv = lax.rsqrt(deg)
    x0_ref[...] = x0
    hs = dinv * x0
    hs0_ref[...] = hs[:, :_HALF]
    hs1_ref[...] = hs[:, _HALF:]


def _tc_prep(x_p, wt, b2, d0, d1):
    grid = (_NPAD // _RT,)
    return pl.pallas_call(
        _prep_body,
        grid=grid,
        in_specs=[
            pl.BlockSpec((_RT, _D), lambda i: (i, 0)),
            pl.BlockSpec((_D, _D), lambda i: (0, 0)),
            pl.BlockSpec((1, _D), lambda i: (0, 0)),
            pl.BlockSpec((_RT, _HALF), lambda i: (i, 0)),
            pl.BlockSpec((_RT, _HALF), lambda i: (i, 0)),
        ],
        out_specs=[
            pl.BlockSpec((_RT, _D), lambda i: (i, 0)),
            pl.BlockSpec((_RT, _HALF), lambda i: (i, 0)),
            pl.BlockSpec((_RT, _HALF), lambda i: (i, 0)),
        ],
        out_shape=[
            jax.ShapeDtypeStruct((_NPAD, _D), jnp.float32),
            jax.ShapeDtypeStruct((_NPAD, _HALF), jnp.float32),
            jax.ShapeDtypeStruct((_NPAD, _HALF), jnp.float32),
        ],
    )(x_p, wt, b2, d0, d1)


def _update_body(beta, relu, a0_ref, a1_ref, x0_ref, h_ref, d0_ref, d1_ref,
                 w_ref, hn_ref, hs0_ref, hs1_ref):
    deg = d0_ref[...][:, 0:1] + d1_ref[...][:, 0:1] + 1.0
    dinv = lax.rsqrt(deg)
    h = h_ref[...]
    aggf = jnp.concatenate([a0_ref[...], a1_ref[...]], axis=1)
    agg = dinv * aggf + (dinv * dinv) * h
    t = (1.0 - _ALPHA) * agg + _ALPHA * x0_ref[...]
    raw = (1.0 - beta) * t + beta * jnp.dot(
        t, w_ref[...], preferred_element_type=jnp.float32)
    h2 = raw + h
    if relu:
        h2 = jnp.maximum(h2, 0.0)
    hn_ref[...] = h2
    hs = dinv * h2
    hs0_ref[...] = hs[:, :_HALF]
    hs1_ref[...] = hs[:, _HALF:]


def _tc_update(a0, a1, x0, h, d0, d1, w, beta, relu):
    grid = (_NPAD // _RT,)
    return pl.pallas_call(
        functools.partial(_update_body, beta, relu),
        grid=grid,
        in_specs=[
            pl.BlockSpec((_RT, _HALF), lambda i: (i, 0)),
            pl.BlockSpec((_RT, _HALF), lambda i: (i, 0)),
            pl.BlockSpec((_RT, _D), lambda i: (i, 0)),
            pl.BlockSpec((_RT, _D), lambda i: (i, 0)),
            pl.BlockSpec((_RT, _HALF), lambda i: (i, 0)),
            pl.BlockSpec((_RT, _HALF), lambda i: (i, 0)),
            pl.BlockSpec((_D, _D), lambda i: (0, 0)),
        ],
        out_specs=[
            pl.BlockSpec((_RT, _D), lambda i: (i, 0)),
            pl.BlockSpec((_RT, _HALF), lambda i: (i, 0)),
            pl.BlockSpec((_RT, _HALF), lambda i: (i, 0)),
        ],
        out_shape=[
            jax.ShapeDtypeStruct((_NPAD, _D), jnp.float32),
            jax.ShapeDtypeStruct((_NPAD, _HALF), jnp.float32),
            jax.ShapeDtypeStruct((_NPAD, _HALF), jnp.float32),
        ],
    )(a0, a1, x0, h, d0, d1, w)


def kernel(x, edge_index, W_proj, b_proj, W_convs):
    src = edge_index[0]
    dst = edge_index[1]
    pad_p = _EP - _E
    srcp = jnp.concatenate([src, jnp.zeros((pad_p,), jnp.int32)])
    dstp = jnp.concatenate([dst, jnp.full((pad_p,), _NPAD - 1, jnp.int32)])
    src3 = jnp.concatenate([srcp, srcp + _NPAD]).reshape(_NC * _NS, _KP, _G)
    dst3 = dstp.reshape(_NS, _KP * _SPG, _GS)
    pad_d = _ED - _E
    dstd = jnp.concatenate([dst, jnp.full((pad_d,), _NPAD - 1, jnp.int32)])
    dstd3 = dstd.reshape(_NC * _NS, _KD, _GS)
    deg_parts = _deg_sc(dstd3)
    d0, d1 = deg_parts[0], deg_parts[1]

    x_p = jnp.pad(x, ((0, _NPAD - _N), (0, 0)))
    x0, hs0, hs1 = _tc_prep(x_p, W_proj.T, b_proj.reshape(1, _D), d0, d1)
    h = x0
    for i in range(_L):
        hs_flat = jnp.concatenate([hs0, hs1], axis=0)
        agg = _prop_sc(hs_flat, src3, dst3)
        beta = float(np.log(_THETA / (i + 1) + 1.0))
        h, hs0, hs1 = _tc_update(agg[0], agg[1], x0, h, d0, d1, W_convs[i],
                                 beta, relu=(i < _L - 1))
    return h[:_N]
